# baseline (speedup 1.0000x reference)
import jax
import jax.numpy as jnp
from jax import lax
from jax.experimental import pallas as pl
from jax.experimental.pallas import tpu as pltpu
from jax.experimental.pallas import tpu_sc as plsc

_D = 1024
_SEQ = 4096
_BATCH = 4
_ROWS = _BATCH * _SEQ
_NC = 2
_NS = 16
_NW = _NC * _NS
_CHUNK = _ROWS // _NW
_GROWS = 72
_LANES = 16
_UNROLL = 4
_NITER = _D // (_LANES * _UNROLL)


def _tec_body(emb_hbm, out_hbm, table_ref, buf_ref, osem):
    wid = lax.axis_index("s") * _NC + lax.axis_index("c")
    base = wid * _CHUNK
    pos = (wid % (_SEQ // _CHUNK)) * _CHUNK
    pos = pos.astype(jnp.int32)
    m = (pos >= 2048).astype(jnp.int32) + (pos >= 3072).astype(jnp.int32)
    seg_start = (pos == 0) | (pos == 2048) | (pos == 3072)

    pltpu.sync_copy(emb_hbm, table_ref)

    w0 = jnp.full((_LANES,), (m == 0).astype(jnp.float32))
    w1 = jnp.full((_LANES,), (m == 1).astype(jnp.float32))
    w2 = jnp.full((_LANES,), (m == 2).astype(jnp.float32))
    ws = jnp.full((_LANES,), seg_start.astype(jnp.float32))

    def fill_first(i, carry):
        for u in range(_UNROLL):
            dsl = pl.ds((i * _UNROLL + u) * _LANES, _LANES)
            t0 = table_ref[0, dsl]
            t1 = table_ref[1, dsl]
            t2 = table_ref[2, dsl]
            t3 = table_ref[3, dsl]
            t4 = table_ref[4, dsl]
            t5 = table_ref[5, dsl]
            vm = t0 * w0 + t1 * w1 + t2 * w2
            vs = t3 * w0 + t4 * w1 + t5 * w2
            vf = vm + (vs - vm) * ws
            buf_ref[0, dsl] = vf
            for r in range(1, 16):
                buf_ref[r, dsl] = vm
        return carry

    def make_fill(lo, hi):
        def fill(i, carry):
            for u in range(_UNROLL):
                dsl = pl.ds((i * _UNROLL + u) * _LANES, _LANES)
                vm = buf_ref[8, dsl]
                for r in range(lo, hi):
                    buf_ref[r, dsl] = vm
            return carry

        return fill

    copies = []

    lax.fori_loop(0, _NITER, fill_first, 0)
    copies.append(
        pltpu.async_copy(
            buf_ref.at[pl.ds(0, 16)], out_hbm.at[pl.ds(base, 16)], osem
        )
    )
    lax.fori_loop(0, _NITER, make_fill(16, 56), 0)
    copies.append(
        pltpu.async_copy(
            buf_ref.at[pl.ds(8, 48)], out_hbm.at[pl.ds(base + 16, 48)], osem
        )
    )
    lax.fori_loop(0, _NITER, make_fill(56, _GROWS), 0)
    for k in range(7):
        copies.append(
            pltpu.async_copy(
                buf_ref.at[pl.ds(8, 64)],
                out_hbm.at[pl.ds(base + 64 + k * 64, 64)],
                osem,
            )
        )
    for c in copies:
        c.wait()


@jax.jit
def _modal_embed(emb):
    out = pl.kernel(
        _tec_body,
        mesh=plsc.VectorSubcoreMesh(core_axis_name="c", subcore_axis_name="s"),
        out_type=jax.ShapeDtypeStruct((_ROWS, _D), jnp.float32),
        scratch_types=[
            pltpu.VMEM((6, _D), jnp.float32),
            pltpu.VMEM((_GROWS, _D), jnp.float32),
            pltpu.SemaphoreType.DMA,
        ],
    )(emb)
    return out.reshape(_BATCH, _SEQ, _D)


def kernel(modal_feat_0, modal_feat_1, modal_feat_2, modal_emb):
    del modal_feat_0, modal_feat_1, modal_feat_2
    return _modal_embed(modal_emb)


# speedup vs baseline: 1.0409x; 1.0409x over previous
import jax
import jax.numpy as jnp
from jax import lax
from jax.experimental import pallas as pl
from jax.experimental.pallas import tpu as pltpu
from jax.experimental.pallas import tpu_sc as plsc

_D = 1024
_SEQ = 4096
_BATCH = 4
_ROWS = _BATCH * _SEQ
_NC = 2
_NS = 16
_NW = _NC * _NS
_CHUNK = _ROWS // _NW
_GROWS = 72
_LANES = 16
_UNROLL = 2
_NITER = _D // (_LANES * _UNROLL)


def _tec_body(emb_hbm, out_hbm, table_ref, buf_ref, osem):
    wid = lax.axis_index("s") * _NC + lax.axis_index("c")
    base = wid * _CHUNK
    pos = (wid % (_SEQ // _CHUNK)) * _CHUNK
    pos = pos.astype(jnp.int32)
    m = (pos >= 2048).astype(jnp.int32) + (pos >= 3072).astype(jnp.int32)
    seg_start = (pos == 0) | (pos == 2048) | (pos == 3072)

    pltpu.sync_copy(emb_hbm, table_ref)

    w0 = jnp.full((_LANES,), (m == 0).astype(jnp.float32))
    w1 = jnp.full((_LANES,), (m == 1).astype(jnp.float32))
    w2 = jnp.full((_LANES,), (m == 2).astype(jnp.float32))
    ws = jnp.full((_LANES,), seg_start.astype(jnp.float32))

    def fill_first(i, carry):
        for u in range(_UNROLL):
            dsl = pl.ds((i * _UNROLL + u) * _LANES, _LANES)
            t0 = table_ref[0, dsl]
            t1 = table_ref[1, dsl]
            t2 = table_ref[2, dsl]
            t3 = table_ref[3, dsl]
            t4 = table_ref[4, dsl]
            t5 = table_ref[5, dsl]
            vm = t0 * w0 + t1 * w1 + t2 * w2
            vs = t3 * w0 + t4 * w1 + t5 * w2
            vf = vm + (vs - vm) * ws
            buf_ref[0, dsl] = vf
            for r in range(1, 16):
                buf_ref[r, dsl] = vm
        return carry

    def make_fill(lo, hi):
        def fill(i, carry):
            for u in range(_UNROLL):
                dsl = pl.ds((i * _UNROLL + u) * _LANES, _LANES)
                vm = buf_ref[8, dsl]
                for r in range(lo, hi):
                    buf_ref[r, dsl] = vm
            return carry

        return fill

    copies = []

    lax.fori_loop(0, _NITER, fill_first, 0)
    copies.append(
        pltpu.async_copy(
            buf_ref.at[pl.ds(0, 16)], out_hbm.at[pl.ds(base, 16)], osem
        )
    )
    lax.fori_loop(0, _NITER, make_fill(16, 40), 0)
    copies.append(
        pltpu.async_copy(
            buf_ref.at[pl.ds(8, 32)], out_hbm.at[pl.ds(base + 16, 32)], osem
        )
    )
    lax.fori_loop(0, _NITER, make_fill(40, _GROWS), 0)
    copies.append(
        pltpu.async_copy(
            buf_ref.at[pl.ds(8, 64)], out_hbm.at[pl.ds(base + 48, 64)], osem
        )
    )
    for k in range(6):
        copies.append(
            pltpu.async_copy(
                buf_ref.at[pl.ds(8, 64)],
                out_hbm.at[pl.ds(base + 112 + k * 64, 64)],
                osem,
            )
        )
    copies.append(
        pltpu.async_copy(
            buf_ref.at[pl.ds(8, 16)], out_hbm.at[pl.ds(base + 496, 16)], osem
        )
    )
    for c in copies:
        c.wait()


@jax.jit
def _modal_embed(emb):
    out = pl.kernel(
        _tec_body,
        mesh=plsc.VectorSubcoreMesh(core_axis_name="c", subcore_axis_name="s"),
        out_type=jax.ShapeDtypeStruct((_ROWS, _D), jnp.float32),
        scratch_types=[
            pltpu.VMEM((6, _D), jnp.float32),
            pltpu.VMEM((_GROWS, _D), jnp.float32),
            pltpu.SemaphoreType.DMA,
        ],
    )(emb)
    return out.reshape(_BATCH, _SEQ, _D)


def kernel(modal_feat_0, modal_feat_1, modal_feat_2, modal_emb):
    del modal_feat_0, modal_feat_1, modal_feat_2
    return _modal_embed(modal_emb)


# speedup vs baseline: 1.0463x; 1.0052x over previous
import jax
import jax.numpy as jnp
from jax import lax
from jax.experimental import pallas as pl
from jax.experimental.pallas import tpu as pltpu
from jax.experimental.pallas import tpu_sc as plsc

_D = 1024
_SEQ = 4096
_BATCH = 4
_ROWS = _BATCH * _SEQ
_NC = 2
_NS = 16
_NW = _NC * _NS
_CHUNK = _ROWS // _NW
_GROWS = 72
_LANES = 16
_UNROLL = 1
_NITER = _D // (_LANES * _UNROLL)


def _tec_body(emb_hbm, out_hbm, table_ref, buf_ref, osem):
    wid = lax.axis_index("s") * _NC + lax.axis_index("c")
    base = wid * _CHUNK
    pos = (wid % (_SEQ // _CHUNK)) * _CHUNK
    pos = pos.astype(jnp.int32)
    m = (pos >= 2048).astype(jnp.int32) + (pos >= 3072).astype(jnp.int32)
    seg_start = (pos == 0) | (pos == 2048) | (pos == 3072)

    pltpu.sync_copy(emb_hbm, table_ref)

    w0 = jnp.full((_LANES,), (m == 0).astype(jnp.float32))
    w1 = jnp.full((_LANES,), (m == 1).astype(jnp.float32))
    w2 = jnp.full((_LANES,), (m == 2).astype(jnp.float32))
    ws = jnp.full((_LANES,), seg_start.astype(jnp.float32))

    def fill_first(i, carry):
        for u in range(_UNROLL):
            dsl = pl.ds((i * _UNROLL + u) * _LANES, _LANES)
            t0 = table_ref[0, dsl]
            t1 = table_ref[1, dsl]
            t2 = table_ref[2, dsl]
            t3 = table_ref[3, dsl]
            t4 = table_ref[4, dsl]
            t5 = table_ref[5, dsl]
            vm = t0 * w0 + t1 * w1 + t2 * w2
            vs = t3 * w0 + t4 * w1 + t5 * w2
            vf = vm + (vs - vm) * ws
            buf_ref[0, dsl] = vf
            for r in range(1, 16):
                buf_ref[r, dsl] = vm
        return carry

    def make_fill(lo, hi):
        def fill(i, carry):
            for u in range(_UNROLL):
                dsl = pl.ds((i * _UNROLL + u) * _LANES, _LANES)
                vm = buf_ref[8, dsl]
                for r in range(lo, hi):
                    buf_ref[r, dsl] = vm
            return carry

        return fill

    copies = []

    lax.fori_loop(0, _NITER, fill_first, 0)
    copies.append(
        pltpu.async_copy(
            buf_ref.at[pl.ds(0, 16)], out_hbm.at[pl.ds(base, 16)], osem
        )
    )
    lax.fori_loop(0, _NITER, make_fill(16, 56), 0)
    copies.append(
        pltpu.async_copy(
            buf_ref.at[pl.ds(8, 48)], out_hbm.at[pl.ds(base + 16, 48)], osem
        )
    )
    lax.fori_loop(0, _NITER, make_fill(56, _GROWS), 0)
    for k in range(7):
        copies.append(
            pltpu.async_copy(
                buf_ref.at[pl.ds(8, 64)],
                out_hbm.at[pl.ds(base + 64 + k * 64, 64)],
                osem,
            )
        )
    for c in copies:
        c.wait()


@jax.jit
def _modal_embed(emb):
    out = pl.kernel(
        _tec_body,
        mesh=plsc.VectorSubcoreMesh(core_axis_name="c", subcore_axis_name="s"),
        out_type=jax.ShapeDtypeStruct((_ROWS, _D), jnp.float32),
        scratch_types=[
            pltpu.VMEM((6, _D), jnp.float32),
            pltpu.VMEM((_GROWS, _D), jnp.float32),
            pltpu.SemaphoreType.DMA,
        ],
    )(emb)
    return out.reshape(_BATCH, _SEQ, _D)


def kernel(modal_feat_0, modal_feat_1, modal_feat_2, modal_emb):
    del modal_feat_0, modal_feat_1, modal_feat_2
    return _modal_embed(modal_emb)


# speedup vs baseline: 1.1173x; 1.0679x over previous
import jax
import jax.numpy as jnp
from jax import lax
from jax.experimental import pallas as pl
from jax.experimental.pallas import tpu as pltpu
from jax.experimental.pallas import tpu_sc as plsc

_D = 1024
_SEQ = 4096
_BATCH = 4
_ROWS = _BATCH * _SEQ
_NC = 2
_NS = 16
_NW = _NC * _NS
_CHUNK = _ROWS // _NW
_GROWS = 72
_LANES = 16
_NITER = _D // _LANES


def _tec_body(emb_hbm, out_hbm, buf_ref, lsem, osem):
    wid = lax.axis_index("s") * _NC + lax.axis_index("c")
    base = wid * _CHUNK
    pos = (wid % (_SEQ // _CHUNK)) * _CHUNK
    pos = pos.astype(jnp.int32)
    m = (pos >= 2048).astype(jnp.int32) + (pos >= 3072).astype(jnp.int32)
    seg_start = (pos == 0) | (pos == 2048) | (pos == 3072)
    first = m + 3 * seg_start.astype(jnp.int32)

    c1 = pltpu.async_copy(
        emb_hbm.at[pl.ds(first, 1)], buf_ref.at[pl.ds(0, 1)], lsem
    )
    c2 = pltpu.async_copy(emb_hbm.at[pl.ds(m, 1)], buf_ref.at[pl.ds(8, 1)], lsem)
    c1.wait()
    c2.wait()

    def make_fill(lo, hi):
        def fill(c, carry):
            dsl = pl.ds(c * _LANES, _LANES)
            vm = buf_ref[8, dsl]
            for r in range(lo, hi):
                if r != 8:
                    buf_ref[r, dsl] = vm
            return carry

        return fill

    copies = []

    lax.fori_loop(0, _NITER, make_fill(1, 16), 0)
    copies.append(
        pltpu.async_copy(
            buf_ref.at[pl.ds(0, 16)], out_hbm.at[pl.ds(base, 16)], osem
        )
    )
    lax.fori_loop(0, _NITER, make_fill(16, 40), 0)
    copies.append(
        pltpu.async_copy(
            buf_ref.at[pl.ds(8, 32)], out_hbm.at[pl.ds(base + 16, 32)], osem
        )
    )
    lax.fori_loop(0, _NITER, make_fill(40, _GROWS), 0)
    copies.append(
        pltpu.async_copy(
            buf_ref.at[pl.ds(8, 64)], out_hbm.at[pl.ds(base + 48, 64)], osem
        )
    )
    for k in range(6):
        copies.append(
            pltpu.async_copy(
                buf_ref.at[pl.ds(8, 64)],
                out_hbm.at[pl.ds(base + 112 + k * 64, 64)],
                osem,
            )
        )
    copies.append(
        pltpu.async_copy(
            buf_ref.at[pl.ds(8, 16)], out_hbm.at[pl.ds(base + 496, 16)], osem
        )
    )
    for c in copies:
        c.wait()


@jax.jit
def _modal_embed(emb):
    out = pl.kernel(
        _tec_body,
        mesh=plsc.VectorSubcoreMesh(core_axis_name="c", subcore_axis_name="s"),
        out_type=jax.ShapeDtypeStruct((_ROWS, _D), jnp.float32),
        scratch_types=[
            pltpu.VMEM((_GROWS, _D), jnp.float32),
            pltpu.SemaphoreType.DMA,
            pltpu.SemaphoreType.DMA,
        ],
    )(emb)
    return out.reshape(_BATCH, _SEQ, _D)


def kernel(modal_feat_0, modal_feat_1, modal_feat_2, modal_emb):
    del modal_feat_0, modal_feat_1, modal_feat_2
    return _modal_embed(modal_emb)
